# Initial kernel scaffold; baseline (speedup 1.0000x reference)
import jax
import jax.numpy as jnp
from jax.experimental import pallas as pl


def kernel(input):
    raise NotImplementedError("write your pallas kernel here")


# speedup vs baseline: 32.8660x; 32.8660x over previous
import functools

import jax
import jax.numpy as jnp
from jax import lax
from jax.experimental import pallas as pl
from jax.experimental.pallas import tpu as pltpu
from jax.experimental.pallas import tpu_sc as plsc

N = 2 * 8192 * 2048
NLV = 127.0
_info = plsc.get_sparse_core_info()
NC, NS, L = _info.num_cores, _info.num_subcores, _info.num_lanes
NW = NC * NS
CHUNK = N // NW
T = 16384
TILES = CHUNK // T
H1 = 65536
H2 = 32768 + 8
DUMP = 32768

_mesh = plsc.VectorSubcoreMesh(core_axis_name="c", subcore_axis_name="s")


def _zero_fill(ref, nbins):
    z = jnp.zeros((L,), jnp.int32)

    @pl.loop(0, nbins // L, unroll=8)
    def _(j):
        ref[pl.ds(j * L, L)] = z


def _dma(x_hbm, buf_v, base, t, b, sem):
    return pltpu.make_async_copy(
        x_hbm.at[pl.ds(base + t * T, T)], buf_v.at[b], sem)


@functools.partial(
    pl.kernel,
    out_type=[
        jax.ShapeDtypeStruct((NW, H1), jnp.int32),
        jax.ShapeDtypeStruct((NW, L), jnp.int32),
    ],
    mesh=_mesh,
    scratch_types=[
        pltpu.VMEM((H1,), jnp.int32),
        pltpu.VMEM((2, T), jnp.int32),
        pltpu.VMEM((L,), jnp.int32),
        pltpu.SemaphoreType.DMA,
        pltpu.SemaphoreType.DMA,
    ],
    compiler_params=pltpu.CompilerParams(needs_layout_passes=False),
)
def _hist1(x_hbm, hist_out, zc_out, hist_v, buf_v, zc_v, sem0, sem1):
    wid = lax.axis_index("s") * NC + lax.axis_index("c")
    base = wid * CHUNK
    _zero_fill(hist_v, H1)
    ones = jnp.ones((L,), jnp.int32)
    zeros = jnp.zeros((L,), jnp.int32)

    def process(b, zacc):
        def body(j, zc):
            a = buf_v[b, pl.ds(j * L, L)] & jnp.int32(0x7FFFFFFF)
            plsc.addupdate_scatter(hist_v, [a >> 15], ones)
            return zc + jnp.where(a == 0, ones, zeros)
        return pl.loop(0, T // L, init_carry=zacc, unroll=8)(body)

    _dma(x_hbm, buf_v, base, 0, 0, sem0).start()
    _dma(x_hbm, buf_v, base, 1, 1, sem1).start()

    def outer(i, zacc):
        for b, sem in ((0, sem0), (1, sem1)):
            t = 2 * i + b
            _dma(x_hbm, buf_v, base, t, b, sem).wait()
            zacc = process(b, zacc)
            tn = jnp.minimum(t + 2, TILES - 2 + b)
            _dma(x_hbm, buf_v, base, tn, b, sem).start()
        return zacc

    zacc = pl.loop(0, TILES // 2, init_carry=zeros)(outer)
    _dma(x_hbm, buf_v, base, TILES - 2, 0, sem0).wait()
    _dma(x_hbm, buf_v, base, TILES - 1, 1, sem1).wait()

    pltpu.sync_copy(hist_v, hist_out.at[wid])
    zc_v[...] = zacc
    pltpu.sync_copy(zc_v, zc_out.at[wid])


@functools.partial(
    pl.kernel,
    out_type=jax.ShapeDtypeStruct((NW, H2), jnp.int32),
    mesh=_mesh,
    scratch_types=[
        pltpu.VMEM((H2,), jnp.int32),
        pltpu.VMEM((2, T), jnp.int32),
        pltpu.VMEM((L,), jnp.int32),
        pltpu.SemaphoreType.DMA,
        pltpu.SemaphoreType.DMA,
    ],
    compiler_params=pltpu.CompilerParams(needs_layout_passes=False),
)
def _hist2(x_hbm, b1_hbm, hist_out, hist_v, buf_v, b1_v, sem0, sem1):
    wid = lax.axis_index("s") * NC + lax.axis_index("c")
    base = wid * CHUNK
    _zero_fill(hist_v, H2)
    pltpu.sync_copy(b1_hbm, b1_v)
    b1vec = b1_v[...]
    ones = jnp.ones((L,), jnp.int32)
    dump = jnp.full((L,), DUMP, jnp.int32)

    def process(b):
        def body(j):
            a = buf_v[b, pl.ds(j * L, L)] & jnp.int32(0x7FFFFFFF)
            bins = jnp.where((a >> 15) == b1vec, a & 0x7FFF, dump)
            plsc.addupdate_scatter(hist_v, [bins], ones)
        pl.loop(0, T // L, unroll=8)(body)

    _dma(x_hbm, buf_v, base, 0, 0, sem0).start()
    _dma(x_hbm, buf_v, base, 1, 1, sem1).start()

    def outer(i):
        for b, sem in ((0, sem0), (1, sem1)):
            t = 2 * i + b
            _dma(x_hbm, buf_v, base, t, b, sem).wait()
            process(b)
            tn = jnp.minimum(t + 2, TILES - 2 + b)
            _dma(x_hbm, buf_v, base, tn, b, sem).start()

    pl.loop(0, TILES // 2)(outer)
    _dma(x_hbm, buf_v, base, TILES - 2, 0, sem0).wait()
    _dma(x_hbm, buf_v, base, TILES - 1, 1, sem1).wait()

    pltpu.sync_copy(hist_v, hist_out.at[wid])


def _quant_body(th_ref, x_ref, o_ref):
    th = th_ref[0, 0]
    safe = jnp.where(th == 0.0, jnp.float32(1.0), th)
    x = x_ref[...]
    clipped = jnp.clip(x, -safe, safe)
    q = jnp.round(clipped / safe * NLV) / NLV * safe
    o_ref[...] = jnp.where(th == 0.0, x, q)


_ROWS, _COLS, _BLK = 16384, 2048, 512

_quant = pl.pallas_call(
    _quant_body,
    grid=(_ROWS // _BLK,),
    in_specs=[
        pl.BlockSpec(memory_space=pltpu.SMEM),
        pl.BlockSpec((_BLK, _COLS), lambda i: (i, 0)),
    ],
    out_specs=pl.BlockSpec((_BLK, _COLS), lambda i: (i, 0)),
    out_shape=jax.ShapeDtypeStruct((_ROWS, _COLS), jnp.float32),
)


@jax.jit
def kernel(input):
    flat = input.reshape(-1)
    flat_i = lax.bitcast_convert_type(flat, jnp.int32)
    h1p, zcp = _hist1(flat_i)
    hist1 = jnp.sum(h1p, axis=0)
    num_zero = jnp.sum(zcp)
    num_nonzero = jnp.int32(N) - num_zero
    idx = (0.9 * num_nonzero).astype(jnp.int32) + (N - num_nonzero).astype(jnp.int32)
    k = jnp.minimum(idx, N - 1)
    c1 = jnp.cumsum(hist1)
    b1 = jnp.sum(c1 <= k).astype(jnp.int32)
    r = k - (c1[b1] - hist1[b1])
    h2p = _hist2(flat_i, jnp.full((L,), b1, jnp.int32))
    hist2 = jnp.sum(h2p, axis=0)[:DUMP]
    c2 = jnp.cumsum(hist2)
    b2 = jnp.sum(c2 <= r).astype(jnp.int32)
    th_bits = (b1 << 15) | b2
    th_val = lax.bitcast_convert_type(th_bits, jnp.float32)
    out = _quant(th_val.reshape(1, 1), flat.reshape(_ROWS, _COLS))
    return out.reshape(input.shape)


# speedup vs baseline: 63.2323x; 1.9239x over previous
import functools

import jax
import jax.numpy as jnp
from jax import lax
from jax.experimental import pallas as pl
from jax.experimental.pallas import tpu as pltpu
from jax.experimental.pallas import tpu_sc as plsc

N = 2 * 8192 * 2048
NLV = 127.0
_info = plsc.get_sparse_core_info()
NC, NS, L = _info.num_cores, _info.num_subcores, _info.num_lanes
NW = NC * NS
CHUNK = N // NW
T = 16384
TILES = CHUNK // T
H1 = 65537 + 7
H2 = 32768

_mesh = plsc.VectorSubcoreMesh(core_axis_name="c", subcore_axis_name="s")
_params = pltpu.CompilerParams(needs_layout_passes=False)


def _zero_fill(ref, nbins):
    z = jnp.zeros((L,), jnp.int32)

    @plsc.parallel_loop(0, nbins // L, unroll=8)
    def _(j):
        ref[pl.ds(j * L, L)] = z


def _dma(x_hbm, buf_v, base, t, b, sem):
    return pltpu.make_async_copy(
        x_hbm.at[pl.ds(base + t * T, T)], buf_v.at[b], sem)


def _stream_tiles(x_hbm, buf_v, base, sem0, sem1, process):
    _dma(x_hbm, buf_v, base, 0, 0, sem0).start()
    _dma(x_hbm, buf_v, base, 1, 1, sem1).start()

    def outer(i):
        for b, sem in ((0, sem0), (1, sem1)):
            t = 2 * i + b
            _dma(x_hbm, buf_v, base, t, b, sem).wait()
            process(b)
            tn = jnp.minimum(t + 2, TILES - 2 + b)
            _dma(x_hbm, buf_v, base, tn, b, sem).start()

    pl.loop(0, TILES // 2)(outer)
    _dma(x_hbm, buf_v, base, TILES - 2, 0, sem0).wait()
    _dma(x_hbm, buf_v, base, TILES - 1, 1, sem1).wait()


@functools.partial(
    pl.kernel,
    out_type=jax.ShapeDtypeStruct((NW, H1), jnp.int32),
    mesh=_mesh,
    scratch_types=[
        pltpu.VMEM((H1,), jnp.int32),
        pltpu.VMEM((2, T), jnp.float32),
        pltpu.SemaphoreType.DMA,
        pltpu.SemaphoreType.DMA,
    ],
    compiler_params=_params,
)
def _hist1(x_hbm, hist_out, hist_v, buf_v, sem0, sem1):
    wid = lax.axis_index("s") * NC + lax.axis_index("c")
    base = wid * CHUNK
    _zero_fill(hist_v, H1)
    ones = jnp.ones((L,), jnp.int32)

    def process(b):
        @plsc.parallel_loop(0, T // L, unroll=8)
        def _(j):
            u = plsc.bitcast(buf_v[b, pl.ds(j * L, L)], jnp.int32)
            bins = ((u & jnp.int32(0x7FFFFFFF)) + jnp.int32(0x7FFF)) >> 15
            plsc.addupdate_scatter(hist_v, [bins], ones)

    _stream_tiles(x_hbm, buf_v, base, sem0, sem1, process)
    pltpu.sync_copy(hist_v, hist_out.at[wid])


@functools.partial(
    pl.kernel,
    out_type=jax.ShapeDtypeStruct((NW, H2), jnp.int32),
    mesh=_mesh,
    scratch_types=[
        pltpu.VMEM((H2,), jnp.int32),
        pltpu.VMEM((2, T), jnp.float32),
        pltpu.VMEM((L,), jnp.int32),
        pltpu.SemaphoreType.DMA,
        pltpu.SemaphoreType.DMA,
    ],
    compiler_params=_params,
)
def _hist2(x_hbm, base_hbm, hist_out, hist_v, buf_v, base_v, sem0, sem1):
    wid = lax.axis_index("s") * NC + lax.axis_index("c")
    base = wid * CHUNK
    _zero_fill(hist_v, H2)
    pltpu.sync_copy(base_hbm, base_v)
    basevec = base_v[...]
    ones = jnp.ones((L,), jnp.int32)

    def process(b):
        @plsc.parallel_loop(0, T // L, unroll=8)
        def _(j):
            u = plsc.bitcast(buf_v[b, pl.ds(j * L, L)], jnp.int32)
            t = (u & jnp.int32(0x7FFFFFFF)) - basevec
            m = plsc.bitcast(t, jnp.uint32) < jnp.uint32(H2)
            plsc.addupdate_scatter(
                hist_v, [t & jnp.int32(H2 - 1)], ones, mask=m)

    _stream_tiles(x_hbm, buf_v, base, sem0, sem1, process)
    pltpu.sync_copy(hist_v, hist_out.at[wid])


def _quant_body(th_ref, x_ref, o_ref):
    th = th_ref[0, 0]
    safe = jnp.where(th == 0.0, jnp.float32(1.0), th)
    x = x_ref[...]
    clipped = jnp.clip(x, -safe, safe)
    q = jnp.round(clipped / safe * NLV) / NLV * safe
    o_ref[...] = jnp.where(th == 0.0, x, q)


_ROWS, _COLS, _BLK = 16384, 2048, 512

_quant = pl.pallas_call(
    _quant_body,
    grid=(_ROWS // _BLK,),
    in_specs=[
        pl.BlockSpec(memory_space=pltpu.SMEM),
        pl.BlockSpec((_BLK, _COLS), lambda i: (i, 0)),
    ],
    out_specs=pl.BlockSpec((_BLK, _COLS), lambda i: (i, 0)),
    out_shape=jax.ShapeDtypeStruct((_ROWS, _COLS), jnp.float32),
)


@jax.jit
def kernel(input):
    flat = input.reshape(-1)
    h1p = _hist1(flat)
    hist1 = jnp.sum(h1p, axis=0)[:65537]
    num_zero = hist1[0]
    num_nonzero = jnp.int32(N) - num_zero
    idx = (0.9 * num_nonzero).astype(jnp.int32) + (N - num_nonzero).astype(jnp.int32)
    k = jnp.minimum(idx, N - 1)
    c1 = jnp.cumsum(hist1)
    b1 = jnp.sum(c1 <= k).astype(jnp.int32)
    r = k - (c1[b1] - hist1[b1])
    bin_base = ((b1 - 1) << 15) + 1
    h2p = _hist2(flat, jnp.full((L,), bin_base, jnp.int32))
    hist2 = jnp.sum(h2p, axis=0)
    c2 = jnp.cumsum(hist2)
    b2 = jnp.sum(c2 <= r).astype(jnp.int32)
    th_bits = jnp.where(b1 == 0, jnp.int32(0), bin_base + b2)
    th_val = lax.bitcast_convert_type(th_bits, jnp.float32)
    out = _quant(th_val.reshape(1, 1), flat.reshape(_ROWS, _COLS))
    return out.reshape(input.shape)


# speedup vs baseline: 204.9166x; 3.2407x over previous
import functools

import jax
import jax.numpy as jnp
from jax import lax
from jax.experimental import pallas as pl
from jax.experimental.pallas import tpu as pltpu
from jax.experimental.pallas import tpu_sc as plsc

N = 2 * 8192 * 2048
NLV = 127.0
_info = plsc.get_sparse_core_info()
NC, NS, L = _info.num_cores, _info.num_subcores, _info.num_lanes
NW = NC * NS
_ROWS, _COLS = 16384, 2048
RPW = _ROWS // NW
TR = 8
T = TR * _COLS
TILES = RPW // TR
VPR = _COLS // L
H1 = 65537 + 7
H2 = 32768

_mesh = plsc.VectorSubcoreMesh(core_axis_name="c", subcore_axis_name="s")
_params = pltpu.CompilerParams(needs_layout_passes=False)


def _zero_fill(ref, nbins):
    z = jnp.zeros((L,), jnp.int32)

    @plsc.parallel_loop(0, nbins // L, unroll=8)
    def _(j):
        ref[pl.ds(j * L, L)] = z


def _dma(x_hbm, buf_v, base, t, b, sem):
    return pltpu.make_async_copy(
        x_hbm.at[pl.ds(base + t * TR, TR)], buf_v.at[b], sem)


def _stream_tiles(x_hbm, buf_v, base, sem0, sem1, process):
    _dma(x_hbm, buf_v, base, 0, 0, sem0).start()
    _dma(x_hbm, buf_v, base, 1, 1, sem1).start()

    def outer(i):
        for b, sem in ((0, sem0), (1, sem1)):
            t = 2 * i + b
            _dma(x_hbm, buf_v, base, t, b, sem).wait()
            process(b)
            tn = jnp.minimum(t + 2, TILES - 2 + b)
            _dma(x_hbm, buf_v, base, tn, b, sem).start()

    pl.loop(0, TILES // 2)(outer)
    _dma(x_hbm, buf_v, base, TILES - 2, 0, sem0).wait()
    _dma(x_hbm, buf_v, base, TILES - 1, 1, sem1).wait()


@functools.partial(
    pl.kernel,
    out_type=jax.ShapeDtypeStruct((NW, H1), jnp.int32),
    mesh=_mesh,
    scratch_types=[
        pltpu.VMEM((H1,), jnp.int32),
        pltpu.VMEM((2, TR, _COLS), jnp.float32),
        pltpu.SemaphoreType.DMA,
        pltpu.SemaphoreType.DMA,
    ],
    compiler_params=_params,
)
def _hist1(x_hbm, hist_out, hist_v, buf_v, sem0, sem1):
    wid = lax.axis_index("s") * NC + lax.axis_index("c")
    base = wid * RPW
    _zero_fill(hist_v, H1)
    ones = jnp.ones((L,), jnp.int32)

    def process(b):
        for r in range(TR):
            @plsc.parallel_loop(0, VPR, unroll=8)
            def _(j, r=r):
                v = buf_v[b, r, pl.ds(j * L, L)]
                u = plsc.bitcast(v, jnp.int32)
                bins = ((u & jnp.int32(0x7FFFFFFF)) + jnp.int32(0x7FFF)) >> 15
                plsc.addupdate_scatter(hist_v, [bins], ones)

    _stream_tiles(x_hbm, buf_v, base, sem0, sem1, process)
    pltpu.sync_copy(hist_v, hist_out.at[wid])


@functools.partial(
    pl.kernel,
    out_type=jax.ShapeDtypeStruct((NW, H2), jnp.int32),
    mesh=_mesh,
    scratch_types=[
        pltpu.VMEM((H2,), jnp.int32),
        pltpu.VMEM((2, TR, _COLS), jnp.float32),
        pltpu.VMEM((L,), jnp.int32),
        pltpu.SemaphoreType.DMA,
        pltpu.SemaphoreType.DMA,
    ],
    compiler_params=_params,
)
def _hist2(x_hbm, base_hbm, hist_out, hist_v, buf_v, base_v, sem0, sem1):
    wid = lax.axis_index("s") * NC + lax.axis_index("c")
    base = wid * RPW
    _zero_fill(hist_v, H2)
    pltpu.sync_copy(base_hbm, base_v)
    basevec = base_v[...]
    ones = jnp.ones((L,), jnp.int32)

    def process(b):
        for r in range(TR):
            @plsc.parallel_loop(0, VPR, unroll=8)
            def _(j, r=r):
                v = buf_v[b, r, pl.ds(j * L, L)]
                u = plsc.bitcast(v, jnp.int32)
                t = (u & jnp.int32(0x7FFFFFFF)) - basevec
                m = plsc.bitcast(t, jnp.uint32) < jnp.uint32(H2)
                plsc.addupdate_scatter(
                    hist_v, [t & jnp.int32(H2 - 1)], ones, mask=m)

    _stream_tiles(x_hbm, buf_v, base, sem0, sem1, process)
    pltpu.sync_copy(hist_v, hist_out.at[wid])


_BLK = 512


def _quant_body(th_ref, x_ref, o_ref):
    th = th_ref[0, 0]
    safe = jnp.where(th == 0.0, jnp.float32(1.0), th)
    x = x_ref[...]
    clipped = jnp.clip(x, -safe, safe)
    q = jnp.round(clipped / safe * NLV) / NLV * safe
    o_ref[...] = jnp.where(th == 0.0, x, q)


_quant = pl.pallas_call(
    _quant_body,
    grid=(_ROWS // _BLK,),
    in_specs=[
        pl.BlockSpec(memory_space=pltpu.SMEM),
        pl.BlockSpec((_BLK, _COLS), lambda i: (i, 0)),
    ],
    out_specs=pl.BlockSpec((_BLK, _COLS), lambda i: (i, 0)),
    out_shape=jax.ShapeDtypeStruct((_ROWS, _COLS), jnp.float32),
)


@jax.jit
def kernel(input):
    x2 = input.reshape(_ROWS, _COLS)
    h1p = _hist1(x2)
    hist1 = jnp.sum(h1p, axis=0)[:65537]
    num_zero = hist1[0]
    num_nonzero = jnp.int32(N) - num_zero
    idx = (0.9 * num_nonzero).astype(jnp.int32) + (N - num_nonzero).astype(jnp.int32)
    k = jnp.minimum(idx, N - 1)
    c1 = jnp.cumsum(hist1)
    b1 = jnp.sum(c1 <= k).astype(jnp.int32)
    r = k - (c1[b1] - hist1[b1])
    bin_base = ((b1 - 1) << 15) + 1
    h2p = _hist2(x2, jnp.full((L,), bin_base, jnp.int32))
    hist2 = jnp.sum(h2p, axis=0)
    c2 = jnp.cumsum(hist2)
    b2 = jnp.sum(c2 <= r).astype(jnp.int32)
    th_bits = jnp.where(b1 == 0, jnp.int32(0), bin_base + b2)
    th_val = lax.bitcast_convert_type(th_bits, jnp.float32)
    out = _quant(th_val.reshape(1, 1), x2)
    return out.reshape(input.shape)


# speedup vs baseline: 206.3086x; 1.0068x over previous
import functools

import jax
import jax.numpy as jnp
from jax import lax
from jax.experimental import pallas as pl
from jax.experimental.pallas import tpu as pltpu
from jax.experimental.pallas import tpu_sc as plsc

N = 2 * 8192 * 2048
NLV = 127.0
_info = plsc.get_sparse_core_info()
NC, NS, L = _info.num_cores, _info.num_subcores, _info.num_lanes
NW = NC * NS
_ROWS, _COLS = 16384, 2048
RPW = _ROWS // NW
TR = 8
T = TR * _COLS
TILES = RPW // TR
VPR = _COLS // L
H1 = 65537 + 7
H2 = 32768

_mesh = plsc.VectorSubcoreMesh(core_axis_name="c", subcore_axis_name="s")
_params = pltpu.CompilerParams(needs_layout_passes=False)


def _zero_fill(ref, nbins):
    z = jnp.zeros((L,), jnp.int32)

    @plsc.parallel_loop(0, nbins // L, unroll=8)
    def _(j):
        ref[pl.ds(j * L, L)] = z


def _dma(x_hbm, buf_v, base, t, b, sem):
    return pltpu.make_async_copy(
        x_hbm.at[pl.ds(base + t * TR, TR)], buf_v.at[b], sem)


def _stream_tiles(x_hbm, buf_v, base, sem0, sem1, process):
    _dma(x_hbm, buf_v, base, 0, 0, sem0).start()
    _dma(x_hbm, buf_v, base, 1, 1, sem1).start()

    def outer(i):
        for b, sem in ((0, sem0), (1, sem1)):
            t = 2 * i + b
            _dma(x_hbm, buf_v, base, t, b, sem).wait()
            process(b)
            tn = jnp.minimum(t + 2, TILES - 2 + b)
            _dma(x_hbm, buf_v, base, tn, b, sem).start()

    pl.loop(0, TILES // 2)(outer)
    _dma(x_hbm, buf_v, base, TILES - 2, 0, sem0).wait()
    _dma(x_hbm, buf_v, base, TILES - 1, 1, sem1).wait()


@functools.partial(
    pl.kernel,
    out_type=jax.ShapeDtypeStruct((NW, H1), jnp.int32),
    mesh=_mesh,
    scratch_types=[
        pltpu.VMEM((H1,), jnp.int32),
        pltpu.VMEM((2, TR, _COLS), jnp.float32),
        pltpu.SemaphoreType.DMA,
        pltpu.SemaphoreType.DMA,
    ],
    compiler_params=_params,
)
def _hist1(x_hbm, hist_out, hist_v, buf_v, sem0, sem1):
    wid = lax.axis_index("s") * NC + lax.axis_index("c")
    base = wid * RPW
    _zero_fill(hist_v, H1)
    ones = jnp.ones((L,), jnp.int32)

    def process(b):
        for r in range(TR):
            @plsc.parallel_loop(0, VPR, unroll=8)
            def _(j, r=r):
                v = buf_v[b, r, pl.ds(j * L, L)]
                u = plsc.bitcast(v, jnp.int32)
                bins = ((u & jnp.int32(0x7FFFFFFF)) + jnp.int32(0x7FFF)) >> 15
                plsc.addupdate_scatter(hist_v, [bins], ones)

    _stream_tiles(x_hbm, buf_v, base, sem0, sem1, process)
    pltpu.sync_copy(hist_v, hist_out.at[wid])


@functools.partial(
    pl.kernel,
    out_type=jax.ShapeDtypeStruct((NW, H2), jnp.int32),
    mesh=_mesh,
    scratch_types=[
        pltpu.VMEM((H2,), jnp.int32),
        pltpu.VMEM((2, TR, _COLS), jnp.float32),
        pltpu.VMEM((L,), jnp.int32),
        pltpu.SemaphoreType.DMA,
        pltpu.SemaphoreType.DMA,
    ],
    compiler_params=_params,
)
def _hist2(x_hbm, base_hbm, hist_out, hist_v, buf_v, base_v, sem0, sem1):
    wid = lax.axis_index("s") * NC + lax.axis_index("c")
    base = wid * RPW
    _zero_fill(hist_v, H2)
    pltpu.sync_copy(base_hbm, base_v)
    basevec = base_v[...]
    ones = jnp.ones((L,), jnp.int32)

    def process(b):
        for r in range(TR):
            @plsc.parallel_loop(0, VPR, unroll=8)
            def _(j, r=r):
                v = buf_v[b, r, pl.ds(j * L, L)]
                u = plsc.bitcast(v, jnp.int32)
                t = (u & jnp.int32(0x7FFFFFFF)) - basevec
                m = plsc.bitcast(t, jnp.uint32) < jnp.uint32(H2)
                plsc.addupdate_scatter(
                    hist_v, [t & jnp.int32(H2 - 1)], ones, mask=m)

    _stream_tiles(x_hbm, buf_v, base, sem0, sem1, process)
    pltpu.sync_copy(hist_v, hist_out.at[wid])


_BLK = 1024


def _quant_body(th_ref, x_ref, o_ref):
    th = th_ref[0, 0]
    safe = jnp.where(th == 0.0, jnp.float32(1.0), th)
    x = x_ref[...]
    clipped = jnp.clip(x, -safe, safe)
    q = jnp.round(clipped / safe * NLV) / NLV * safe
    o_ref[...] = jnp.where(th == 0.0, x, q)


_quant = pl.pallas_call(
    _quant_body,
    grid=(_ROWS // _BLK,),
    in_specs=[
        pl.BlockSpec(memory_space=pltpu.SMEM),
        pl.BlockSpec((_BLK, _COLS), lambda i: (i, 0)),
    ],
    out_specs=pl.BlockSpec((_BLK, _COLS), lambda i: (i, 0)),
    out_shape=jax.ShapeDtypeStruct((_ROWS, _COLS), jnp.float32),
)


@jax.jit
def kernel(input):
    x2 = input.reshape(_ROWS, _COLS)
    h1p = _hist1(x2)
    hist1 = jnp.sum(h1p, axis=0)[:65537]
    num_zero = hist1[0]
    num_nonzero = jnp.int32(N) - num_zero
    idx = (0.9 * num_nonzero).astype(jnp.int32) + (N - num_nonzero).astype(jnp.int32)
    k = jnp.minimum(idx, N - 1)
    c1 = jnp.cumsum(hist1)
    b1 = jnp.sum(c1 <= k).astype(jnp.int32)
    r = k - (c1[b1] - hist1[b1])
    bin_base = ((b1 - 1) << 15) + 1
    h2p = _hist2(x2, jnp.full((L,), bin_base, jnp.int32))
    hist2 = jnp.sum(h2p, axis=0)
    c2 = jnp.cumsum(hist2)
    b2 = jnp.sum(c2 <= r).astype(jnp.int32)
    th_bits = jnp.where(b1 == 0, jnp.int32(0), bin_base + b2)
    th_val = lax.bitcast_convert_type(th_bits, jnp.float32)
    out = _quant(th_val.reshape(1, 1), x2)
    return out.reshape(input.shape)
